# baseline (speedup 1.0000x reference)
import functools

import jax
import jax.numpy as jnp
from jax import lax
from jax.experimental import pallas as pl
from jax.experimental.pallas import tpu as pltpu
from jax.experimental.pallas import tpu_sc as plsc

_KPROB = 0.5
_SCALE = 1.0 / _KPROB
_BLOCK = 896 * 1024


def _dropout_body(v_ref, r_ref, o_ref):
    mask = jnp.floor(r_ref[...] + _KPROB)
    o_ref[...] = v_ref[...] * (mask * _SCALE)


def _tc_dropout(values, rand_vals):
    nnz = values.shape[0]
    grid = pl.cdiv(nnz, _BLOCK)
    return pl.pallas_call(
        _dropout_body,
        grid=(grid,),
        in_specs=[
            pl.BlockSpec((_BLOCK,), lambda i: (i,)),
            pl.BlockSpec((_BLOCK,), lambda i: (i,)),
        ],
        out_specs=pl.BlockSpec((_BLOCK,), lambda i: (i,)),
        out_shape=jax.ShapeDtypeStruct((nnz,), jnp.float32),
    )(values, rand_vals)


@functools.lru_cache(maxsize=None)
def _make_sc_copy(rows: int, cols: int):
    info = plsc.get_sparse_core_info()
    nc, ns = info.num_cores, info.num_subcores
    nw = nc * ns
    wc = 8064
    w = (cols // (nw * 128)) * 128
    tailc = cols - nw * w
    nfull = w // wc
    rem = w - nfull * wc
    nch = nfull + (1 if rem else 0)

    mesh = plsc.VectorSubcoreMesh(core_axis_name="c", subcore_axis_name="s")

    @functools.partial(
        pl.kernel,
        mesh=mesh,
        out_type=jax.ShapeDtypeStruct((rows, cols), jnp.int32),
        scratch_types=[
            pltpu.VMEM_SHARED((ns, 2, rows, wc), jnp.int32),
            pltpu.SemaphoreType.DMA,
            pltpu.SemaphoreType.DMA,
            pltpu.SemaphoreType.DMA,
            pltpu.SemaphoreType.DMA,
        ],
    )
    def sc_copy(src, dst, shared, in0, in1, out0, out1):
        c = lax.axis_index("c")
        s = lax.axis_index("s")
        wid = s * nc + c
        base = wid * w
        in_sems = (in0, in1)
        out_sems = (out0, out1)

        def cwidth(k):
            return wc if k < nfull else rem

        def sbuf(k):
            if cwidth(k) == wc:
                return shared.at[s, k % 2]
            return shared.at[s, k % 2, :, pl.ds(0, cwidth(k))]

        def hslice(ref, k):
            return ref.at[:, pl.ds(base + k * wc, cwidth(k))]

        ins = {}
        outs = {}
        ins[0] = pltpu.async_copy(hslice(src, 0), sbuf(0), in_sems[0])
        for k in range(nch):
            if k + 1 < nch:
                if k - 1 >= 0:
                    outs[k - 1].wait()
                ins[k + 1] = pltpu.async_copy(
                    hslice(src, k + 1), sbuf(k + 1), in_sems[(k + 1) % 2]
                )
            ins[k].wait()
            outs[k] = pltpu.async_copy(
                sbuf(k), hslice(dst, k), out_sems[k % 2]
            )
        for k in (nch - 2, nch - 1):
            if k >= 0:
                outs[k].wait()

        def bounce(col, width):
            sbuf2 = shared.at[s, 0, :, pl.ds(0, width)]
            pltpu.sync_copy(src.at[:, pl.ds(col, width)], sbuf2)
            pltpu.sync_copy(sbuf2, dst.at[:, pl.ds(col, width)])

        if tailc:
            tcol = nw * w
            t128 = (tailc // 128) * 128
            tlast = tailc - t128

            @pl.when(wid == 0)
            def _():
                if t128:
                    bounce(tcol, t128)
                if tlast:
                    def tail_body(tbuf):
                        for r in range(rows):
                            pltpu.sync_copy(
                                src.at[r, pl.ds(tcol + t128, tlast)], tbuf
                            )
                            pltpu.sync_copy(
                                tbuf, dst.at[r, pl.ds(tcol + t128, tlast)]
                            )

                    pl.run_scoped(tail_body, pltpu.VMEM((tlast,), jnp.int32))

    return sc_copy


def _fused_body(i_ref, v_ref, r_ref, oi_ref, ov_ref):
    oi_ref[...] = i_ref[...]
    mask = jnp.floor(r_ref[...] + _KPROB)
    ov_ref[...] = v_ref[...] * (mask * _SCALE)


def _tc_fused(indices, values, rand_vals):
    nnz = values.shape[0]
    b = _BLOCK
    grid = pl.cdiv(nnz, b)
    return pl.pallas_call(
        _fused_body,
        grid=(grid,),
        in_specs=[
            pl.BlockSpec((2, b), lambda i: (0, i)),
            pl.BlockSpec((b,), lambda i: (i,)),
            pl.BlockSpec((b,), lambda i: (i,)),
        ],
        out_specs=[
            pl.BlockSpec((2, b), lambda i: (0, i)),
            pl.BlockSpec((b,), lambda i: (i,)),
        ],
        out_shape=[
            jax.ShapeDtypeStruct((2, nnz), jnp.int32),
            jax.ShapeDtypeStruct((nnz,), jnp.float32),
        ],
    )(indices, values, rand_vals)


def kernel(indices, values, rand_vals):
    idx_out, new_vals = _tc_fused(indices, values, rand_vals)
    return idx_out, new_vals


# speedup vs baseline: 1.0155x; 1.0155x over previous
import jax
import jax.numpy as jnp
from jax.experimental import pallas as pl

_KPROB = 0.5
_SCALE = 1.0 / _KPROB
_BLOCK = 768 * 1024


def _fused_body(i_ref, v_ref, r_ref, oi_ref, ov_ref):
    oi_ref[...] = i_ref[...]
    mask = jnp.floor(r_ref[...] + _KPROB)
    ov_ref[...] = v_ref[...] * (mask * _SCALE)


def _tc_fused(indices, values, rand_vals):
    nnz = values.shape[0]
    b = _BLOCK
    grid = pl.cdiv(nnz, b)
    return pl.pallas_call(
        _fused_body,
        grid=(grid,),
        in_specs=[
            pl.BlockSpec((2, b), lambda i: (0, i)),
            pl.BlockSpec((b,), lambda i: (i,)),
            pl.BlockSpec((b,), lambda i: (i,)),
        ],
        out_specs=[
            pl.BlockSpec((2, b), lambda i: (0, i)),
            pl.BlockSpec((b,), lambda i: (i,)),
        ],
        out_shape=[
            jax.ShapeDtypeStruct((2, nnz), jnp.int32),
            jax.ShapeDtypeStruct((nnz,), jnp.float32),
        ],
    )(indices, values, rand_vals)


def kernel(indices, values, rand_vals):
    idx_out, new_vals = _tc_fused(indices, values, rand_vals)
    return idx_out, new_vals
